# baseline (speedup 1.0000x reference)
import functools

import jax
import jax.numpy as jnp
from jax import lax
from jax.experimental import pallas as pl
from jax.experimental.pallas import tpu as pltpu
from jax.experimental.pallas import tpu_sc as plsc

H = 1024
F = 4096
E = 8
TOP_K = 2
TM = 256
W32 = H // 2


def _router_body(x_ref, wr_ref, a1_ref, a2_ref, w1_ref, w2_ref):
    xb = x_ref[...]
    wr = wr_ref[...]
    logits = lax.dot_general(xb, wr, (((1,), (1,)), ((), ())),
                             preferred_element_type=jnp.float32)
    m = jnp.max(logits, axis=1, keepdims=True)
    ex = jnp.exp(logits - m)
    sm = ex / jnp.sum(ex, axis=1, keepdims=True)
    cols = lax.broadcasted_iota(jnp.int32, sm.shape, 1)
    w1v = jnp.max(sm, axis=1, keepdims=True)
    a1v = jnp.min(jnp.where(sm == w1v, cols, E), axis=1, keepdims=True)
    sm2 = jnp.where(cols == a1v, -1.0, sm)
    w2v = jnp.max(sm2, axis=1, keepdims=True)
    a2v = jnp.min(jnp.where(sm2 == w2v, cols, E), axis=1, keepdims=True)
    a1_ref[...] = a1v
    a2_ref[...] = a2v
    w1_ref[...] = w1v
    w2_ref[...] = w2v


def _router(xf, W_router):
    n = xf.shape[0]
    return pl.pallas_call(
        _router_body,
        out_shape=[
            jax.ShapeDtypeStruct((n, 1), jnp.int32),
            jax.ShapeDtypeStruct((n, 1), jnp.int32),
            jax.ShapeDtypeStruct((n, 1), jnp.float32),
            jax.ShapeDtypeStruct((n, 1), jnp.float32),
        ],
    )(xf, W_router)


def _sc_dispatch(table, idx):
    n_idx = idx.shape[0]
    info = plsc.get_sparse_core_info()
    nw = info.num_cores * info.num_subcores
    rows_per = n_idx // nw
    chunk = 64
    n_ch = rows_per // chunk
    mesh = plsc.VectorSubcoreMesh(core_axis_name="c", subcore_axis_name="s")

    @functools.partial(
        pl.kernel,
        mesh=mesh,
        out_type=jax.ShapeDtypeStruct((n_idx, W32), jnp.int32),
        scratch_types=[
            pltpu.VMEM((rows_per,), jnp.int32),
            pltpu.VMEM((chunk, W32), jnp.int32),
            pltpu.VMEM((chunk, W32), jnp.int32),
            pltpu.SemaphoreType.DMA,
            pltpu.SemaphoreType.DMA,
            pltpu.SemaphoreType.DMA,
            pltpu.SemaphoreType.DMA,
        ],
    )
    def k(tab, idx_hbm, out, idx_v, r0, r1, g0, g1, w0, w1):
        wid = lax.axis_index("s") * info.num_cores + lax.axis_index("c")
        base = wid * rows_per
        pltpu.sync_copy(idx_hbm.at[pl.ds(base, rows_per)], idx_v)
        bufs, gs, ws = (r0, r1), (g0, g1), (w0, w1)
        gcp = [None] * n_ch
        wcp = [None] * n_ch
        gcp[0] = pltpu.async_copy(tab.at[idx_v.at[pl.ds(0, chunk)]],
                                  bufs[0], gs[0])
        for c in range(n_ch):
            b = c % 2
            b2 = (c + 1) % 2
            if c + 1 < n_ch:
                if c >= 1:
                    wcp[c - 1].wait()
                gcp[c + 1] = pltpu.async_copy(
                    tab.at[idx_v.at[pl.ds((c + 1) * chunk, chunk)]],
                    bufs[b2], gs[b2])
            gcp[c].wait()
            wcp[c] = pltpu.async_copy(
                bufs[b], out.at[pl.ds(base + c * chunk, chunk)], ws[b])
        for t in range(max(0, n_ch - 2), n_ch):
            wcp[t].wait()

    return k(table, idx)


def _combine_body(yun_ref, out_ref):
    out_ref[...] = (yun_ref[:, :H].astype(jnp.float32)
                    + yun_ref[:, H:].astype(jnp.float32))


def _combine(yun2):
    n = yun2.shape[0]
    bt = 512
    return pl.pallas_call(
        _combine_body,
        grid=(n // bt,),
        in_specs=[pl.BlockSpec((bt, 2 * H), lambda i: (i, 0))],
        out_specs=pl.BlockSpec((bt, H), lambda i: (i, 0)),
        out_shape=jax.ShapeDtypeStruct((n, H), jnp.float32),
    )(yun2)


def _gemm_body(te_ref, xs_ref, w1_ref, w2_ref, sc_ref, out_ref):
    xb = xs_ref[...]
    pre = lax.dot_general(xb, w1_ref[0], (((1,), (1,)), ((), ())),
                          preferred_element_type=jnp.float32)
    act = jax.nn.gelu(pre, approximate=True).astype(jnp.bfloat16)
    y = lax.dot_general(act, w2_ref[0], (((1,), (0,)), ((), ())),
                        preferred_element_type=jnp.float32)
    out_ref[...] = (y * sc_ref[0]).astype(jnp.bfloat16)


def _grouped_gemm(xs, w1c, w2c, tile_expert, scales3, n_tiles):
    grid_spec = pltpu.PrefetchScalarGridSpec(
        num_scalar_prefetch=1,
        grid=(n_tiles,),
        in_specs=[
            pl.BlockSpec((TM, H), lambda m, te: (m, 0)),
            pl.BlockSpec((1, F, H), lambda m, te: (te[m], 0, 0)),
            pl.BlockSpec((1, F, H), lambda m, te: (te[m], 0, 0)),
            pl.BlockSpec((1, TM, 1), lambda m, te: (m, 0, 0)),
        ],
        out_specs=pl.BlockSpec((TM, H), lambda m, te: (m, 0)),
    )
    return pl.pallas_call(
        _gemm_body,
        grid_spec=grid_spec,
        out_shape=jax.ShapeDtypeStruct((n_tiles * TM, H), jnp.bfloat16),
        compiler_params=pltpu.CompilerParams(
            dimension_semantics=("arbitrary",)),
    )(tile_expert, xs, w1c, w2c, scales3)


def kernel(x, W_router, w1, w2):
    in_shape = x.shape
    xf = x.reshape(-1, H)
    n = xf.shape[0]
    a_tot = n * TOP_K
    pt = a_tot + E * TM
    n_tiles = pt // TM

    a1, a2, wv1, wv2 = _router(xf, W_router)

    e_flat = jnp.stack([a1[:, 0], a2[:, 0]], axis=1).reshape(-1)
    onehot = (e_flat[:, None] == jnp.arange(E)[None, :]).astype(jnp.int32)
    within = jnp.cumsum(onehot, axis=0) - onehot
    rank = jnp.take_along_axis(within, e_flat[:, None], axis=1)[:, 0]
    counts = jnp.sum(onehot, axis=0)
    padded = ((counts + TM - 1) // TM) * TM
    off_dst = jnp.concatenate([jnp.zeros((1,), jnp.int32),
                               jnp.cumsum(padded)[:-1].astype(jnp.int32)])
    dst_a = off_dst[e_flat] + rank
    slot_token = jnp.zeros((pt,), jnp.int32).at[dst_a].set(
        jnp.arange(a_tot, dtype=jnp.int32) // TOP_K)
    w_flat = jnp.stack([wv1[:, 0], wv2[:, 0]], axis=1).reshape(-1)
    slot_scale = jnp.zeros((pt,), jnp.float32).at[dst_a].set(w_flat)
    bounds = jnp.cumsum(padded)
    tile_expert = jnp.clip(
        jnp.searchsorted(bounds, jnp.arange(n_tiles, dtype=jnp.int32) * TM,
                         side="right").astype(jnp.int32), 0, E - 1)

    xi = lax.bitcast_convert_type(
        xf.astype(jnp.bfloat16).reshape(n, W32, 2), jnp.int32)
    xs = lax.bitcast_convert_type(
        _sc_dispatch(xi, slot_token), jnp.bfloat16).reshape(pt, H)

    w1c = w1.astype(jnp.bfloat16).reshape(E, F, H)
    w2c = w2.astype(jnp.bfloat16).reshape(E, F, H)
    ys = _grouped_gemm(xs, w1c, w2c, tile_expert,
                       slot_scale.reshape(n_tiles, TM, 1), n_tiles)

    ys_i = lax.bitcast_convert_type(ys.reshape(pt, W32, 2), jnp.int32)
    yun = lax.bitcast_convert_type(
        _sc_dispatch(ys_i, dst_a), jnp.bfloat16).reshape(n, TOP_K * H)
    out = _combine(yun)
    return out.reshape(in_shape)


# speedup vs baseline: 7.8842x; 7.8842x over previous
import functools

import jax
import jax.numpy as jnp
import numpy as np
from jax import lax
from jax.experimental import pallas as pl
from jax.experimental.pallas import tpu as pltpu
from jax.experimental.pallas import tpu_sc as plsc

H = 1024
F = 4096
E = 8
TOP_K = 2
TM = 256
HW = H // 2
_HI = np.uint32(0xFFFF0000)


def _pack(rows_f32):
    r = rows_f32.astype(jnp.bfloat16).astype(jnp.float32)
    b = lax.bitcast_convert_type(r, jnp.uint32)
    w = (b[:, :HW] >> 16) | (b[:, HW:] & _HI)
    return lax.bitcast_convert_type(w, jnp.int32)


def _unpack(rows_i32):
    wu = lax.bitcast_convert_type(rows_i32, jnp.uint32)
    lo = lax.bitcast_convert_type(wu << 16, jnp.float32)
    hi = lax.bitcast_convert_type(wu & _HI, jnp.float32)
    return jnp.concatenate([lo, hi], axis=1)


def _router_body(x_ref, wr_ref, a1_ref, a2_ref, w1_ref, w2_ref, xi_ref):
    xb = x_ref[...]
    wr = wr_ref[...]
    logits = lax.dot_general(xb, wr, (((1,), (1,)), ((), ())),
                             preferred_element_type=jnp.float32)
    m = jnp.max(logits, axis=1, keepdims=True)
    ex = jnp.exp(logits - m)
    sm = ex / jnp.sum(ex, axis=1, keepdims=True)
    cols = lax.broadcasted_iota(jnp.int32, sm.shape, 1)
    w1v = jnp.max(sm, axis=1, keepdims=True)
    a1v = jnp.min(jnp.where(sm == w1v, cols, E), axis=1, keepdims=True)
    sm2 = jnp.where(cols == a1v, -1.0, sm)
    w2v = jnp.max(sm2, axis=1, keepdims=True)
    a2v = jnp.min(jnp.where(sm2 == w2v, cols, E), axis=1, keepdims=True)
    a1_ref[...] = a1v
    a2_ref[...] = a2v
    w1_ref[...] = w1v
    w2_ref[...] = w2v
    xi_ref[...] = _pack(xb)


def _router(xf, W_router):
    n = xf.shape[0]
    return pl.pallas_call(
        _router_body,
        out_shape=[
            jax.ShapeDtypeStruct((n, 1), jnp.int32),
            jax.ShapeDtypeStruct((n, 1), jnp.int32),
            jax.ShapeDtypeStruct((n, 1), jnp.float32),
            jax.ShapeDtypeStruct((n, 1), jnp.float32),
            jax.ShapeDtypeStruct((n, HW), jnp.int32),
        ],
    )(xf, W_router)


def _sc_dispatch(table, idx):
    n_idx = idx.shape[0]
    info = plsc.get_sparse_core_info()
    nw = info.num_cores * info.num_subcores
    rows_per = n_idx // nw
    chunk = 64
    n_ch = rows_per // chunk
    mesh = plsc.VectorSubcoreMesh(core_axis_name="c", subcore_axis_name="s")

    @functools.partial(
        pl.kernel,
        mesh=mesh,
        out_type=jax.ShapeDtypeStruct((n_idx, HW), jnp.int32),
        scratch_types=[
            pltpu.VMEM((rows_per,), jnp.int32),
            pltpu.VMEM((chunk, HW), jnp.int32),
            pltpu.VMEM((chunk, HW), jnp.int32),
            pltpu.SemaphoreType.DMA,
            pltpu.SemaphoreType.DMA,
            pltpu.SemaphoreType.DMA,
            pltpu.SemaphoreType.DMA,
        ],
    )
    def k(tab, idx_hbm, out, idx_v, r0, r1, g0, g1, w0, w1):
        wid = lax.axis_index("s") * info.num_cores + lax.axis_index("c")
        base = wid * rows_per
        pltpu.sync_copy(idx_hbm.at[pl.ds(base, rows_per)], idx_v)
        bufs, gs, ws = (r0, r1), (g0, g1), (w0, w1)
        gcp = [None] * n_ch
        wcp = [None] * n_ch
        gcp[0] = pltpu.async_copy(tab.at[idx_v.at[pl.ds(0, chunk)]],
                                  bufs[0], gs[0])
        for c in range(n_ch):
            b = c % 2
            b2 = (c + 1) % 2
            if c + 1 < n_ch:
                if c >= 1:
                    wcp[c - 1].wait()
                gcp[c + 1] = pltpu.async_copy(
                    tab.at[idx_v.at[pl.ds((c + 1) * chunk, chunk)]],
                    bufs[b2], gs[b2])
            gcp[c].wait()
            wcp[c] = pltpu.async_copy(
                bufs[b], out.at[pl.ds(base + c * chunk, chunk)], ws[b])
        for t in range(max(0, n_ch - 2), n_ch):
            wcp[t].wait()

    return k(table, idx)


def _combine_body(yun_ref, out_ref):
    wu = lax.bitcast_convert_type(yun_ref[...], jnp.uint32)
    pa, pb = wu[:, :HW], wu[:, HW:]
    a_lo = lax.bitcast_convert_type(pa << 16, jnp.float32)
    a_hi = lax.bitcast_convert_type(pa & _HI, jnp.float32)
    b_lo = lax.bitcast_convert_type(pb << 16, jnp.float32)
    b_hi = lax.bitcast_convert_type(pb & _HI, jnp.float32)
    out_ref[...] = jnp.concatenate([a_lo + b_lo, a_hi + b_hi], axis=1)


def _combine(yun2):
    n = yun2.shape[0]
    bt = 512
    return pl.pallas_call(
        _combine_body,
        grid=(n // bt,),
        in_specs=[pl.BlockSpec((bt, 2 * HW), lambda i: (i, 0))],
        out_specs=pl.BlockSpec((bt, H), lambda i: (i, 0)),
        out_shape=jax.ShapeDtypeStruct((n, H), jnp.float32),
    )(yun2)


def _gemm_body(te_ref, xs_ref, w1_ref, w2_ref, sc_ref, out_ref):
    xb = _unpack(xs_ref[...]).astype(jnp.bfloat16)
    pre = lax.dot_general(xb, w1_ref[0], (((1,), (1,)), ((), ())),
                          preferred_element_type=jnp.float32)
    act = jax.nn.gelu(pre, approximate=True).astype(jnp.bfloat16)
    y = lax.dot_general(act, w2_ref[0], (((1,), (0,)), ((), ())),
                        preferred_element_type=jnp.float32)
    out_ref[...] = _pack(y * sc_ref[0])


def _grouped_gemm(xs, w1c, w2c, tile_expert, scales3, n_tiles):
    grid_spec = pltpu.PrefetchScalarGridSpec(
        num_scalar_prefetch=1,
        grid=(n_tiles,),
        in_specs=[
            pl.BlockSpec((TM, HW), lambda m, te: (m, 0)),
            pl.BlockSpec((1, F, H), lambda m, te: (te[m], 0, 0)),
            pl.BlockSpec((1, F, H), lambda m, te: (te[m], 0, 0)),
            pl.BlockSpec((1, TM, 1), lambda m, te: (m, 0, 0)),
        ],
        out_specs=pl.BlockSpec((TM, HW), lambda m, te: (m, 0)),
    )
    return pl.pallas_call(
        _gemm_body,
        grid_spec=grid_spec,
        out_shape=jax.ShapeDtypeStruct((n_tiles * TM, HW), jnp.int32),
        compiler_params=pltpu.CompilerParams(
            dimension_semantics=("arbitrary",)),
    )(tile_expert, xs, w1c, w2c, scales3)


def kernel(x, W_router, w1, w2):
    in_shape = x.shape
    xf = x.reshape(-1, H)
    n = xf.shape[0]
    a_tot = n * TOP_K
    pt = a_tot + E * TM
    n_tiles = pt // TM

    a1, a2, wv1, wv2, xi = _router(xf, W_router)

    e_flat = jnp.stack([a1[:, 0], a2[:, 0]], axis=1).reshape(-1)
    onehot = (e_flat[:, None] == jnp.arange(E)[None, :]).astype(jnp.int32)
    within = jnp.cumsum(onehot, axis=0) - onehot
    rank = jnp.take_along_axis(within, e_flat[:, None], axis=1)[:, 0]
    counts = jnp.sum(onehot, axis=0)
    padded = ((counts + TM - 1) // TM) * TM
    off_dst = jnp.concatenate([jnp.zeros((1,), jnp.int32),
                               jnp.cumsum(padded)[:-1].astype(jnp.int32)])
    dst_a = off_dst[e_flat] + rank
    slot_token = jnp.zeros((pt,), jnp.int32).at[dst_a].set(
        jnp.arange(a_tot, dtype=jnp.int32) // TOP_K)
    w_flat = jnp.stack([wv1[:, 0], wv2[:, 0]], axis=1).reshape(-1)
    slot_scale = jnp.zeros((pt,), jnp.float32).at[dst_a].set(w_flat)
    bounds = jnp.cumsum(padded)
    tile_expert = jnp.clip(
        jnp.searchsorted(bounds, jnp.arange(n_tiles, dtype=jnp.int32) * TM,
                         side="right").astype(jnp.int32), 0, E - 1)

    xs = _sc_dispatch(xi, slot_token)

    w1c = w1.astype(jnp.bfloat16).reshape(E, F, H)
    w2c = w2.astype(jnp.bfloat16).reshape(E, F, H)
    ys = _grouped_gemm(xs, w1c, w2c, tile_expert,
                       slot_scale.reshape(n_tiles, TM, 1), n_tiles)

    yun = _sc_dispatch(ys, dst_a).reshape(n, 2 * HW)
    out = _combine(yun)
    return out.reshape(in_shape)
